# baseline (speedup 1.0000x reference)
import dataclasses
import functools

import jax
import jax.numpy as jnp
from jax import lax
from jax.experimental import pallas as pl
from jax.experimental.pallas import tpu as pltpu
from jax.experimental.pallas import tpu_sc as plsc

B, D, O = 8192, 4096, 4096
RP = 512
CB = 512
BP = B + RP
NRP = BP // RP
NC = O // CB
GRP = 512
GW = 8

_MESH = plsc.VectorSubcoreMesh(core_axis_name="c", subcore_axis_name="s")

_SC_PARAMS = pltpu.CompilerParams()
if "needs_layout_passes" in pltpu.CompilerParams.__dataclass_fields__:
    _SC_PARAMS = dataclasses.replace(_SC_PARAMS, needs_layout_passes=False)


def _gate_body(bg_ref, x_ref, wg_ref, o_ref):
    g = jax.lax.dot_general(
        x_ref[...], wg_ref[...], (((1,), (0,)), ((), ())),
        preferred_element_type=jnp.float32,
    ) + bg_ref[0]
    o_ref[...] = jnp.broadcast_to(g, (GRP, 128))


def _gate(X, wg2, bg):
    return pl.pallas_call(
        _gate_body,
        grid=(B // GRP,),
        in_specs=[
            pl.BlockSpec(memory_space=pltpu.SMEM),
            pl.BlockSpec((GRP, D), lambda i: (i, 0)),
            pl.BlockSpec((D, 1), lambda i: (0, 0)),
        ],
        out_specs=pl.BlockSpec((GRP, 128), lambda i: (i, 0)),
        out_shape=jax.ShapeDtypeStruct((B, 128), jnp.float32),
    )(bg, X, wg2)


def _route(gate):
    @functools.partial(
        pl.kernel,
        out_type=(
            jax.ShapeDtypeStruct((BP,), jnp.int32),
            jax.ShapeDtypeStruct((B,), jnp.int32),
            jax.ShapeDtypeStruct((16,), jnp.int32),
        ),
        mesh=_MESH,
        compiler_params=_SC_PARAMS,
        scratch_types=[
            pltpu.VMEM((B,), jnp.float32),
            pltpu.VMEM((BP,), jnp.int32),
            pltpu.VMEM((B,), jnp.int32),
            pltpu.VMEM((16,), jnp.int32),
            pltpu.SMEM((4,), jnp.int32),
            pltpu.SemaphoreType.DMA,
        ],
    )
    def route_kernel(gate_hbm, perm_hbm, dst_hbm, meta_hbm,
                     gate_v, perm_v, dst_v, meta_v, cnt_s, sem):
        cid = lax.axis_index("c")
        sid = lax.axis_index("s")

        @pl.when(jnp.logical_and(cid == 0, sid == 0))
        def _():
            pltpu.async_copy(gate_hbm, gate_v, sem).wait()

            @pl.loop(0, BP // 16)
            def _(i):
                perm_v[pl.ds(i * 16, 16)] = jnp.zeros((16,), jnp.int32)

            cnt_s[0] = 0

            @pl.loop(0, B // 16)
            def _(i):
                m = (gate_v[pl.ds(i * 16, 16)] > 0.0).astype(jnp.int32)
                cnt_s[0] = cnt_s[0] + jnp.sum(m)

            n_exit = cnt_s[0]
            n_pad = ((n_exit + RP - 1) // RP) * RP
            cnt_s[3] = n_pad
            cnt_s[1] = 0
            cnt_s[2] = 0

            @pl.loop(0, B // 16)
            def _(i):
                g = gate_v[pl.ds(i * 16, 16)]
                m = g > 0.0
                mi = m.astype(jnp.int32)
                exc_e = plsc.cumsum(mi) - mi
                nmi = 1 - mi
                exc_f = plsc.cumsum(nmi) - nmi
                dste = cnt_s[1] + exc_e
                dstf = cnt_s[3] + cnt_s[2] + exc_f
                dstv = jnp.where(m, dste, dstf)
                dst_v[pl.ds(i * 16, 16)] = dstv
                srcv = i * 16 + lax.iota(jnp.int32, 16)
                plsc.store_scatter(perm_v, [dstv], srcv)
                ne = jnp.sum(mi)
                cnt_s[1] = cnt_s[1] + ne
                cnt_s[2] = cnt_s[2] + (16 - ne)

            meta_v[...] = jnp.full((16,), cnt_s[3] // RP, jnp.int32)
            pltpu.async_copy(perm_v, perm_hbm, sem).wait()
            pltpu.async_copy(dst_v, dst_hbm, sem).wait()
            pltpu.async_copy(meta_v, meta_hbm, sem).wait()

    return route_kernel(gate)


def _gather_rows(src, idx, n_rows, n_cols, dtype):
    per_w = n_rows // 32
    n_chunks = per_w // GW

    @functools.partial(
        pl.kernel,
        out_type=jax.ShapeDtypeStruct((n_rows, n_cols), dtype),
        mesh=_MESH,
        compiler_params=_SC_PARAMS,
        scratch_types=[
            pltpu.VMEM((per_w,), jnp.int32),
            pltpu.VMEM((GW, n_cols), dtype),
            pltpu.VMEM((GW, n_cols), dtype),
            pltpu.SemaphoreType.DMA,
            pltpu.SemaphoreType.DMA,
            pltpu.SemaphoreType.DMA,
        ],
    )
    def gather_kernel(src_hbm, idx_hbm, out_hbm,
                      idx_v, buf0, buf1, gsem0, gsem1, wsem):
        wid = lax.axis_index("s") * 2 + lax.axis_index("c")
        base = wid * per_w
        pltpu.sync_copy(idx_hbm.at[pl.ds(base, per_w)], idx_v)

        def start(c, buf, sem):
            pltpu.make_async_copy(
                src_hbm.at[idx_v.at[pl.ds(c * GW, GW)]], buf, sem).start()

        def finish(c, buf, sem):
            pltpu.make_async_copy(
                src_hbm.at[idx_v.at[pl.ds(c * GW, GW)]], buf, sem).wait()
            pltpu.make_async_copy(
                buf, out_hbm.at[pl.ds(base + c * GW, GW)], wsem).start()
            pltpu.make_async_copy(
                buf, out_hbm.at[pl.ds(base + c * GW, GW)], wsem).wait()

        start(0, buf0, gsem0)

        @pl.loop(0, n_chunks, step=2)
        def _(c):
            @pl.when(c + 1 < n_chunks)
            def _():
                start(c + 1, buf1, gsem1)

            finish(c, buf0, gsem0)

            @pl.when(c + 2 < n_chunks)
            def _():
                start(c + 2, buf0, gsem0)

            @pl.when(c + 1 < n_chunks)
            def _():
                finish(c + 1, buf1, gsem1)

    return gather_kernel(src, idx)


def _mm_body(meta_ref, x_ref, wc_ref, bc_ref, wm_ref, bm_ref, o_ref):
    r = pl.program_id(0)
    ncls = meta_ref[0]
    xb = x_ref[...]

    @pl.when(r < ncls)
    def _():
        o_ref[...] = jax.lax.dot_general(
            xb, wc_ref[...], (((1,), (0,)), ((), ())),
            preferred_element_type=jnp.float32,
        ) + bc_ref[0:1, :]

    @pl.when(r >= ncls)
    def _():
        o_ref[...] = jax.lax.dot_general(
            xb, wm_ref[...], (((1,), (0,)), ((), ())),
            preferred_element_type=jnp.float32,
        ) + bm_ref[0:1, :]


def _matmul(meta1, Xp, Wc16, bc8, Wm16, bm8):
    grid_spec = pltpu.PrefetchScalarGridSpec(
        num_scalar_prefetch=1,
        grid=(NRP, NC),
        in_specs=[
            pl.BlockSpec((RP, D), lambda r, c, m: (r, 0)),
            pl.BlockSpec((D, CB), lambda r, c, m: (0, jnp.where(r < m[0], c, 0))),
            pl.BlockSpec((8, CB), lambda r, c, m: (0, jnp.where(r < m[0], c, 0))),
            pl.BlockSpec((D, CB), lambda r, c, m: (0, jnp.where(r < m[0], 0, c))),
            pl.BlockSpec((8, CB), lambda r, c, m: (0, jnp.where(r < m[0], 0, c))),
        ],
        out_specs=pl.BlockSpec((RP, CB), lambda r, c, m: (r, c)),
    )
    return pl.pallas_call(
        _mm_body,
        grid_spec=grid_spec,
        out_shape=jax.ShapeDtypeStruct((BP, O), jnp.float32),
    )(meta1, Xp, Wc16, bc8, Wm16, bm8)


def kernel(X, w_gate, b_gate, W_cls, b_cls, W_mod, b_mod):
    wg2 = w_gate.reshape(D, 1).astype(jnp.bfloat16)
    bg = b_gate.reshape(1)
    bc8 = jnp.broadcast_to(b_cls.reshape(1, O), (8, O))
    bm8 = jnp.broadcast_to(b_mod.reshape(1, O), (8, O))
    Wc16 = W_cls.astype(jnp.bfloat16)
    Wm16 = W_mod.astype(jnp.bfloat16)
    Xb = X.astype(jnp.bfloat16)

    gate2d = _gate(Xb, wg2, bg)
    gate = gate2d[:, 0]
    perm, dst, meta = _route(gate)
    Xb32 = jax.lax.bitcast_convert_type(Xb.reshape(B, D // 2, 2), jnp.int32)
    Xp32 = _gather_rows(Xb32, perm, BP, D // 2, jnp.int32)
    Xp = jax.lax.bitcast_convert_type(Xp32, jnp.bfloat16).reshape(BP, D)
    Yp = _matmul(meta[:1], Xp, Wc16, bc8, Wm16, bm8)
    y = _gather_rows(Yp, dst, B, O, jnp.float32)
    return y


# speedup vs baseline: 2.5559x; 2.5559x over previous
import dataclasses
import functools

import jax
import jax.numpy as jnp
from jax import lax
from jax.experimental import pallas as pl
from jax.experimental.pallas import tpu as pltpu
from jax.experimental.pallas import tpu_sc as plsc

B, D, O = 8192, 4096, 4096
RP = 512
CB = 512
BP = B + RP
NRP = BP // RP
NC = O // CB
GRP = 512
GW = 8

_MESH = plsc.VectorSubcoreMesh(core_axis_name="c", subcore_axis_name="s")

_SC_PARAMS = pltpu.CompilerParams()
if "needs_layout_passes" in pltpu.CompilerParams.__dataclass_fields__:
    _SC_PARAMS = dataclasses.replace(_SC_PARAMS, needs_layout_passes=False)


def _gate_body(bg_ref, x_ref, wg_ref, o_ref):
    g = jax.lax.dot_general(
        x_ref[...].astype(jnp.bfloat16), wg_ref[...], (((1,), (0,)), ((), ())),
        preferred_element_type=jnp.float32,
    ) + bg_ref[0]
    o_ref[...] = jnp.broadcast_to(g, (GRP, 128))


def _gate(X, wg2, bg):
    return pl.pallas_call(
        _gate_body,
        grid=(B // GRP,),
        in_specs=[
            pl.BlockSpec(memory_space=pltpu.SMEM),
            pl.BlockSpec((GRP, D), lambda i: (i, 0)),
            pl.BlockSpec((D, 1), lambda i: (0, 0)),
        ],
        out_specs=pl.BlockSpec((GRP, 128), lambda i: (i, 0)),
        out_shape=jax.ShapeDtypeStruct((B, 128), jnp.float32),
    )(bg, X, wg2)


def _route(gate):
    @functools.partial(
        pl.kernel,
        out_type=(
            jax.ShapeDtypeStruct((BP,), jnp.int32),
            jax.ShapeDtypeStruct((B,), jnp.int32),
            jax.ShapeDtypeStruct((16,), jnp.int32),
        ),
        mesh=_MESH,
        compiler_params=_SC_PARAMS,
        scratch_types=[
            pltpu.VMEM((B,), jnp.float32),
            pltpu.VMEM((BP,), jnp.int32),
            pltpu.VMEM((B,), jnp.int32),
            pltpu.VMEM((16,), jnp.int32),
            pltpu.SMEM((4,), jnp.int32),
            pltpu.SemaphoreType.DMA,
        ],
    )
    def route_kernel(gate_hbm, perm_hbm, dst_hbm, meta_hbm,
                     gate_v, perm_v, dst_v, meta_v, cnt_s, sem):
        cid = lax.axis_index("c")
        sid = lax.axis_index("s")

        @pl.when(jnp.logical_and(cid == 0, sid == 0))
        def _():
            pltpu.async_copy(gate_hbm, gate_v, sem).wait()

            @pl.loop(0, BP // 16)
            def _(i):
                perm_v[pl.ds(i * 16, 16)] = jnp.zeros((16,), jnp.int32)

            cnt_s[0] = 0

            @pl.loop(0, B // 16)
            def _(i):
                m = (gate_v[pl.ds(i * 16, 16)] > 0.0).astype(jnp.int32)
                cnt_s[0] = cnt_s[0] + jnp.sum(m)

            n_exit = cnt_s[0]
            n_pad = ((n_exit + RP - 1) // RP) * RP
            cnt_s[3] = n_pad
            cnt_s[1] = 0
            cnt_s[2] = 0

            @pl.loop(0, B // 16)
            def _(i):
                g = gate_v[pl.ds(i * 16, 16)]
                m = g > 0.0
                mi = m.astype(jnp.int32)
                exc_e = plsc.cumsum(mi) - mi
                nmi = 1 - mi
                exc_f = plsc.cumsum(nmi) - nmi
                dste = cnt_s[1] + exc_e
                dstf = cnt_s[3] + cnt_s[2] + exc_f
                dstv = jnp.where(m, dste, dstf)
                dst_v[pl.ds(i * 16, 16)] = dstv
                srcv = i * 16 + lax.iota(jnp.int32, 16)
                plsc.store_scatter(perm_v, [dstv], srcv)
                ne = jnp.sum(mi)
                cnt_s[1] = cnt_s[1] + ne
                cnt_s[2] = cnt_s[2] + (16 - ne)

            meta_v[...] = jnp.full((16,), cnt_s[3] // RP, jnp.int32)
            pltpu.async_copy(perm_v, perm_hbm, sem).wait()
            pltpu.async_copy(dst_v, dst_hbm, sem).wait()
            pltpu.async_copy(meta_v, meta_hbm, sem).wait()

    return route_kernel(gate)


def _gather_rows(src, idx, n_rows, n_cols, dtype):
    per_w = n_rows // 32
    n_chunks = per_w // GW

    @functools.partial(
        pl.kernel,
        out_type=jax.ShapeDtypeStruct((n_rows, n_cols), dtype),
        mesh=_MESH,
        compiler_params=_SC_PARAMS,
        scratch_types=[
            pltpu.VMEM((per_w,), jnp.int32),
            pltpu.VMEM((GW, n_cols), dtype),
            pltpu.VMEM((GW, n_cols), dtype),
            pltpu.SemaphoreType.DMA,
            pltpu.SemaphoreType.DMA,
            pltpu.SemaphoreType.DMA,
        ],
    )
    def gather_kernel(src_hbm, idx_hbm, out_hbm,
                      idx_v, buf0, buf1, gsem0, gsem1, wsem):
        wid = lax.axis_index("s") * 2 + lax.axis_index("c")
        base = wid * per_w
        pltpu.sync_copy(idx_hbm.at[pl.ds(base, per_w)], idx_v)

        def start(c, buf, sem):
            pltpu.make_async_copy(
                src_hbm.at[idx_v.at[pl.ds(c * GW, GW)]], buf, sem).start()

        def finish(c, buf, sem):
            pltpu.make_async_copy(
                src_hbm.at[idx_v.at[pl.ds(c * GW, GW)]], buf, sem).wait()
            pltpu.make_async_copy(
                buf, out_hbm.at[pl.ds(base + c * GW, GW)], wsem).start()
            pltpu.make_async_copy(
                buf, out_hbm.at[pl.ds(base + c * GW, GW)], wsem).wait()

        start(0, buf0, gsem0)

        @pl.loop(0, n_chunks, step=2)
        def _(c):
            @pl.when(c + 1 < n_chunks)
            def _():
                start(c + 1, buf1, gsem1)

            finish(c, buf0, gsem0)

            @pl.when(c + 2 < n_chunks)
            def _():
                start(c + 2, buf0, gsem0)

            @pl.when(c + 1 < n_chunks)
            def _():
                finish(c + 1, buf1, gsem1)

    return gather_kernel(src, idx)


def _mm_body(meta_ref, x_ref, wc_ref, bc_ref, wm_ref, bm_ref, o_ref, xb_ref):
    r = pl.program_id(0)
    c = pl.program_id(1)
    ncls = meta_ref[0]

    @pl.when(c == 0)
    def _():
        xb_ref[...] = x_ref[...].astype(jnp.bfloat16)

    xb = xb_ref[...]

    @pl.when(r < ncls)
    def _():
        o_ref[...] = jax.lax.dot_general(
            xb, wc_ref[...], (((1,), (0,)), ((), ())),
            preferred_element_type=jnp.float32,
        ) + bc_ref[0:1, :]

    @pl.when(r >= ncls)
    def _():
        o_ref[...] = jax.lax.dot_general(
            xb, wm_ref[...], (((1,), (0,)), ((), ())),
            preferred_element_type=jnp.float32,
        ) + bm_ref[0:1, :]


def _matmul(meta1, Xp, Wc16, bc8, Wm16, bm8):
    grid_spec = pltpu.PrefetchScalarGridSpec(
        num_scalar_prefetch=1,
        grid=(NRP, NC),
        in_specs=[
            pl.BlockSpec((RP, D), lambda r, c, m: (r, 0)),
            pl.BlockSpec((D, CB), lambda r, c, m: (0, jnp.where(r < m[0], c, 0))),
            pl.BlockSpec((8, CB), lambda r, c, m: (0, jnp.where(r < m[0], c, 0))),
            pl.BlockSpec((D, CB), lambda r, c, m: (0, jnp.where(r < m[0], 0, c))),
            pl.BlockSpec((8, CB), lambda r, c, m: (0, jnp.where(r < m[0], 0, c))),
        ],
        out_specs=pl.BlockSpec((RP, CB), lambda r, c, m: (r, c)),
        scratch_shapes=[pltpu.VMEM((RP, D), jnp.bfloat16)],
    )
    return pl.pallas_call(
        _mm_body,
        grid_spec=grid_spec,
        out_shape=jax.ShapeDtypeStruct((BP, O), jnp.float32),
    )(meta1, Xp, Wc16, bc8, Wm16, bm8)


def kernel(X, w_gate, b_gate, W_cls, b_cls, W_mod, b_mod):
    wg2 = w_gate.reshape(D, 1).astype(jnp.bfloat16)
    bg = b_gate.reshape(1)
    bc8 = jnp.broadcast_to(b_cls.reshape(1, O), (8, O))
    bm8 = jnp.broadcast_to(b_mod.reshape(1, O), (8, O))
    Wc16 = W_cls.astype(jnp.bfloat16)
    Wm16 = W_mod.astype(jnp.bfloat16)
    Xb = X.astype(jnp.bfloat16)

    gate2d = _gate(X, wg2, bg)
    gate = gate2d[:, 0]
    perm, dst, meta = _route(gate)
    Xp = _gather_rows(X, perm, BP, D, jnp.float32)
    Yp = _matmul(meta[:1], Xp, Wc16, bc8, Wm16, bm8)
    y = _gather_rows(Yp, dst, B, O, jnp.float32)
    return y
